# Initial kernel scaffold; baseline (speedup 1.0000x reference)
import jax
import jax.numpy as jnp
from jax.experimental import pallas as pl


def kernel(x, weight):
    raise NotImplementedError("write your pallas kernel here")


# speedup vs baseline: 3.1757x; 3.1757x over previous
import functools

import jax
import jax.numpy as jnp
from jax import lax
from jax.experimental import pallas as pl
from jax.experimental.pallas import tpu as pltpu

_CONTRACT_LAST = (((1,), (1,)), ((), ()))


def _mm_block_kernel(x_ref, w_ref, o_ref):
    o_ref[...] = lax.dot_general(
        x_ref[...], w_ref[...], _CONTRACT_LAST,
        preferred_element_type=jnp.float32,
    ).astype(o_ref.dtype)


def _pick_tile(dim, pref):
    t = min(pref, ((dim + 127) // 128) * 128)
    return t


@functools.partial(jax.jit, static_argnames=())
def _linear_bf16(x2d, weight):
    M, K = x2d.shape
    N = weight.shape[0]

    tm = _pick_tile(M, 1024)
    tn = _pick_tile(N, 1024)
    M_pad = ((M + tm - 1) // tm) * tm
    N_pad = ((N + tn - 1) // tn) * tn

    xb = x2d.astype(jnp.bfloat16)
    wb = weight.astype(jnp.bfloat16)
    if M_pad != M:
        xb = jnp.pad(xb, ((0, M_pad - M), (0, 0)))
    if N_pad != N:
        wb = jnp.pad(wb, ((0, N_pad - N), (0, 0)))

    y2d = pl.pallas_call(
        _mm_block_kernel,
        out_shape=jax.ShapeDtypeStruct((M_pad, N_pad), jnp.float32),
        grid=(M_pad // tm, N_pad // tn),
        in_specs=[
            pl.BlockSpec((tm, K), lambda i, j: (i, 0)),
            pl.BlockSpec((tn, K), lambda i, j: (j, 0)),
        ],
        out_specs=pl.BlockSpec((tm, tn), lambda i, j: (i, j)),
        compiler_params=pltpu.CompilerParams(
            dimension_semantics=("parallel", "parallel"),
            vmem_limit_bytes=56 * 1024 * 1024,
        ),
    )(xb, wb)

    if (M_pad, N_pad) != (M, N):
        y2d = y2d[:M, :N]
    return y2d


@jax.jit
def kernel(x, weight):
    *lead, in_dim = x.shape
    out_dim, in_dim_w = weight.shape
    assert in_dim == in_dim_w, (in_dim, in_dim_w)
    M = 1
    for d in lead:
        M *= d
    y2d = _linear_bf16(x.reshape(M, in_dim), weight)
    return y2d.reshape(*lead, out_dim)


# speedup vs baseline: 3.4043x; 1.0720x over previous
import functools

import jax
import jax.numpy as jnp
from jax import lax
from jax.experimental import pallas as pl
from jax.experimental.pallas import tpu as pltpu

_CONTRACT_LAST = (((1,), (1,)), ((), ()))


def _mm_block_kernel(x_ref, w_ref, o_ref):
    o_ref[...] = lax.dot_general(
        x_ref[...].astype(jnp.bfloat16), w_ref[...], _CONTRACT_LAST,
        preferred_element_type=jnp.float32,
    ).astype(o_ref.dtype)


def _pick_tile(dim, pref):
    t = min(pref, ((dim + 127) // 128) * 128)
    return t


@functools.partial(jax.jit, static_argnames=())
def _linear_bf16(x2d, weight):
    M, K = x2d.shape
    N = weight.shape[0]

    tm = _pick_tile(M, 1024)
    tn = _pick_tile(N, 512)
    M_pad = ((M + tm - 1) // tm) * tm
    N_pad = ((N + tn - 1) // tn) * tn

    xb = x2d
    wb = weight.astype(jnp.bfloat16)
    if M_pad != M:
        xb = jnp.pad(xb, ((0, M_pad - M), (0, 0)))
    if N_pad != N:
        wb = jnp.pad(wb, ((0, N_pad - N), (0, 0)))

    y2d = pl.pallas_call(
        _mm_block_kernel,
        out_shape=jax.ShapeDtypeStruct((M_pad, N_pad), jnp.float32),
        grid=(M_pad // tm, N_pad // tn),
        in_specs=[
            pl.BlockSpec((tm, K), lambda i, j: (i, 0)),
            pl.BlockSpec((tn, K), lambda i, j: (j, 0)),
        ],
        out_specs=pl.BlockSpec((tm, tn), lambda i, j: (i, j)),
        compiler_params=pltpu.CompilerParams(
            dimension_semantics=("parallel", "parallel"),
            vmem_limit_bytes=60 * 1024 * 1024,
        ),
    )(xb, wb)

    if (M_pad, N_pad) != (M, N):
        y2d = y2d[:M, :N]
    return y2d


@jax.jit
def kernel(x, weight):
    *lead, in_dim = x.shape
    out_dim, in_dim_w = weight.shape
    assert in_dim == in_dim_w, (in_dim, in_dim_w)
    M = 1
    for d in lead:
        M *= d
    y2d = _linear_bf16(x.reshape(M, in_dim), weight)
    return y2d.reshape(*lead, out_dim)


# speedup vs baseline: 3.9111x; 1.1489x over previous
import jax
import jax.numpy as jnp
from jax import lax
from jax.experimental import pallas as pl
from jax.experimental.pallas import tpu as pltpu

_CONTRACT_LAST = (((1,), (1,)), ((), ()))


def _mm_block_kernel(x_ref, w_ref, o_ref):
    o_ref[...] = lax.dot_general(
        x_ref[...].astype(jnp.bfloat16),
        w_ref[...].astype(jnp.bfloat16),
        _CONTRACT_LAST,
        preferred_element_type=jnp.float32,
    ).astype(o_ref.dtype)


def _pick_tile(dim, pref):
    return min(pref, ((dim + 127) // 128) * 128)


def _linear_bf16(x2d, weight):
    M, K = x2d.shape
    N = weight.shape[0]

    tm = _pick_tile(M, 1024)
    tn = _pick_tile(N, 512)
    M_pad = ((M + tm - 1) // tm) * tm
    N_pad = ((N + tn - 1) // tn) * tn

    xp = x2d
    wp = weight
    if M_pad != M:
        xp = jnp.pad(xp, ((0, M_pad - M), (0, 0)))
    if N_pad != N:
        wp = jnp.pad(wp, ((0, N_pad - N), (0, 0)))

    y2d = pl.pallas_call(
        _mm_block_kernel,
        out_shape=jax.ShapeDtypeStruct((M_pad, N_pad), jnp.float32),
        grid=(M_pad // tm, N_pad // tn),
        in_specs=[
            pl.BlockSpec((tm, K), lambda i, j: (i, 0)),
            pl.BlockSpec((tn, K), lambda i, j: (j, 0)),
        ],
        out_specs=pl.BlockSpec((tm, tn), lambda i, j: (i, j)),
        compiler_params=pltpu.CompilerParams(
            dimension_semantics=("parallel", "parallel"),
            vmem_limit_bytes=60 * 1024 * 1024,
        ),
    )(xp, wp)

    if (M_pad, N_pad) != (M, N):
        y2d = y2d[:M, :N]
    return y2d


@jax.jit
def kernel(x, weight):
    *lead, in_dim = x.shape
    out_dim, in_dim_w = weight.shape
    assert in_dim == in_dim_w, (in_dim, in_dim_w)
    M = 1
    for d in lead:
        M *= d
    y2d = _linear_bf16(x.reshape(M, in_dim), weight)
    return y2d.reshape(*lead, out_dim)
